# baseline (speedup 1.0000x reference)
import functools

import jax
import jax.numpy as jnp
from jax import lax
from jax.experimental import pallas as pl
from jax.experimental.pallas import tpu as pltpu
from jax.experimental.pallas import tpu_sc as plsc

_D = 64
_B = 16384
_NC = 2
_NS = 16
_NW = _NC * _NS
_BPW = _B // _NW
_RPW = 2 * _BPW
_CHUNK = 128
_NCHUNK = _RPW // _CHUNK

_mesh = plsc.VectorSubcoreMesh(core_axis_name="c", subcore_axis_name="s")


@functools.partial(
    pl.kernel,
    mesh=_mesh,
    compiler_params=pltpu.CompilerParams(use_tc_tiling_on_sc=False),
    out_type=jax.ShapeDtypeStruct((_B * 16,), jnp.float32),
    scratch_types=[
        pltpu.VMEM((_NCHUNK, _CHUNK), jnp.int32),
        pltpu.VMEM((_RPW, _D), jnp.float32),
        pltpu.VMEM((_BPW * 16,), jnp.float32),
        pltpu.SemaphoreType.DMA,
    ],
)
def _sc_pair_lorentz(idx_hbm, w_hbm, out_hbm, idx_v, rows_v, t_v, sem):
    wid = lax.axis_index("s") * _NC + lax.axis_index("c")
    pltpu.sync_copy(idx_hbm.at[wid], idx_v)

    copies = [
        pltpu.async_copy(
            w_hbm.at[idx_v.at[i]], rows_v.at[pl.ds(i * _CHUNK, _CHUNK)], sem
        )
        for i in range(_NCHUNK)
    ]
    for c in copies:
        c.wait()

    lane = lax.iota(jnp.int32, 16)
    sgn = jnp.where(lane == 0, jnp.float32(1.0), jnp.float32(-1.0))

    def pair_body(g, carry):
        for k in range(16):
            p = g * 16 + k
            u_row = 2 * p
            v_row = u_row + 1
            t = rows_v[u_row, pl.ds(0, 16)] * rows_v[v_row, pl.ds(0, 16)]
            t = t * sgn
            for q in range(1, 4):
                t = t - (
                    rows_v[u_row, pl.ds(q * 16, 16)]
                    * rows_v[v_row, pl.ds(q * 16, 16)]
                )
            t_v[pl.ds(p * 16, 16)] = t
        return carry

    lax.fori_loop(0, _BPW // 16, pair_body, 0)
    pltpu.sync_copy(t_v, out_hbm.at[pl.ds(wid * _BPW * 16, _BPW * 16)])


def _fold_acosh_body(x_ref, s_ref, o_ref):
    x = x_ref[...]
    arg = jnp.sum(x.reshape(x.shape[0], 8, 16), axis=2)
    arg = jnp.maximum(arg, jnp.float32(1.0 + 1e-9))
    scale = jnp.exp(s_ref[0])
    o_ref[...] = jnp.log(arg + jnp.sqrt((arg - 1.0) * (arg + 1.0))) * scale


@jax.jit
def kernel(idx, w, scale_log):
    idx32 = idx.astype(jnp.int32).reshape(_NW, _NCHUNK, _CHUNK)
    partials = _sc_pair_lorentz(idx32, w)
    out = pl.pallas_call(
        _fold_acosh_body,
        out_shape=jax.ShapeDtypeStruct((_B // 8, 8), jnp.float32),
        in_specs=[
            pl.BlockSpec(memory_space=pltpu.VMEM),
            pl.BlockSpec(memory_space=pltpu.SMEM),
        ],
        out_specs=pl.BlockSpec(memory_space=pltpu.VMEM),
    )(partials.reshape(_B // 8, 128), scale_log)
    return out.reshape(_B)


# speedup vs baseline: 1.0103x; 1.0103x over previous
import functools

import jax
import jax.numpy as jnp
from jax import lax
from jax.experimental import pallas as pl
from jax.experimental.pallas import tpu as pltpu
from jax.experimental.pallas import tpu_sc as plsc

_D = 64
_B = 16384
_NC = 2
_NS = 16
_NW = _NC * _NS
_BPW = _B // _NW
_RPW = 2 * _BPW
_CHUNK = 128
_NCHUNK = _RPW // _CHUNK
_NBUF = 3

_mesh = plsc.VectorSubcoreMesh(core_axis_name="c", subcore_axis_name="s")


@functools.partial(
    pl.kernel,
    mesh=_mesh,
    out_type=jax.ShapeDtypeStruct((_B * 16,), jnp.float32),
    scratch_types=[
        pltpu.VMEM((_NCHUNK, _CHUNK), jnp.int32),
        pltpu.VMEM((_NCHUNK, _CHUNK), jnp.int32),
        pltpu.VMEM((_NBUF, _CHUNK, 2 * _D), jnp.float32),
        pltpu.VMEM((_BPW * 16,), jnp.float32),
        pltpu.SemaphoreType.DMA,
    ],
)
def _sc_pair_lorentz(idx_hbm, w128_hbm, out_hbm, idx_v, half_v, buf_v, t_v,
                     sem):
    wid = lax.axis_index("s") * _NC + lax.axis_index("c")
    pltpu.sync_copy(idx_hbm.at[wid], idx_v)

    for c in range(_NCHUNK):
        for j in range(_CHUNK // 16):
            sl = pl.ds(j * 16, 16)
            half_v[c, sl] = jax.lax.shift_right_logical(idx_v[c, sl], 1)

    def fire(c):
        return pltpu.async_copy(
            w128_hbm.at[half_v.at[c]], buf_v.at[c % _NBUF], sem
        )

    copies = {c: fire(c) for c in range(min(2, _NCHUNK))}

    lane = lax.iota(jnp.int32, 16)
    sgn = jnp.where(lane == 0, jnp.float32(1.0), jnp.float32(-1.0))

    for c in range(_NCHUNK):
        if c + 2 < _NCHUNK:
            copies[c + 2] = fire(c + 2)
        copies[c].wait()
        bu = buf_v.at[c % _NBUF]

        def grp(g, carry, c=c, bu=bu):
            ia = idx_v[c, pl.ds(g * 32, 16)]
            ib = idx_v[c, pl.ds(g * 32 + 16, 16)]
            for kk in range(16):
                k = g * 16 + kk
                src = ia if kk < 8 else ib
                iu = src[(2 * kk) % 16]
                iv = src[(2 * kk + 1) % 16]
                offu = (iu & 1) * _D
                offv = (iv & 1) * _D
                t = (
                    bu[2 * k, pl.ds(offu, 16)]
                    * bu[2 * k + 1, pl.ds(offv, 16)]
                    * sgn
                )
                for q in range(1, 4):
                    t = t - (
                        bu[2 * k, pl.ds(offu + q * 16, 16)]
                        * bu[2 * k + 1, pl.ds(offv + q * 16, 16)]
                    )
                t_v[pl.ds(c * 1024 + g * 256 + kk * 16, 16)] = t
            return carry

        lax.fori_loop(0, 4, grp, 0)

    pltpu.sync_copy(t_v, out_hbm.at[pl.ds(wid * _BPW * 16, _BPW * 16)])


def _fold_acosh_body(x_ref, s_ref, o_ref):
    x = x_ref[...]
    col = jax.lax.broadcasted_iota(jnp.int32, (128, 8), 0)
    grp = jax.lax.broadcasted_iota(jnp.int32, (128, 8), 1)
    m = jnp.where(col // 16 == grp, jnp.float32(1.0), jnp.float32(0.0))
    arg = jax.lax.dot_general(
        x, m, (((1,), (0,)), ((), ())), preferred_element_type=jnp.float32
    )
    arg = jnp.maximum(arg, jnp.float32(1.0 + 1e-9))
    scale = jnp.exp(s_ref[0])
    o_ref[...] = jnp.log(arg + jnp.sqrt((arg - 1.0) * (arg + 1.0))) * scale


@jax.jit
def kernel(idx, w, scale_log):
    idx32 = idx.astype(jnp.int32).reshape(_NW, _NCHUNK, _CHUNK)
    w128 = w.reshape(-1, 2 * _D)
    partials = _sc_pair_lorentz(idx32, w128)
    out = pl.pallas_call(
        _fold_acosh_body,
        out_shape=jax.ShapeDtypeStruct((_B // 8, 8), jnp.float32),
        in_specs=[
            pl.BlockSpec(memory_space=pltpu.VMEM),
            pl.BlockSpec(memory_space=pltpu.SMEM),
        ],
        out_specs=pl.BlockSpec(memory_space=pltpu.VMEM),
    )(partials.reshape(_B // 8, 128), scale_log)
    return out.reshape(_B)


# speedup vs baseline: 1.7140x; 1.6965x over previous
import functools

import jax
import jax.numpy as jnp
from jax import lax
from jax.experimental import pallas as pl
from jax.experimental.pallas import tpu as pltpu
from jax.experimental.pallas import tpu_sc as plsc

_N = 1_000_000
_D = 64
_B = 16384
_NC = 2
_NS = 16
_NW = _NC * _NS
_BPW = _B // _NW
_RPW = 2 * _BPW
_CHUNK = 128
_NCHUNK = _RPW // _CHUNK
_NBUF = 3
_TBLK = 2048
_TGRID = 245
_HROWS = _TGRID * _TBLK

_mesh = plsc.VectorSubcoreMesh(core_axis_name="c", subcore_axis_name="s")


def _transpose_body(a_ref, o_ref):
    lo = a_ref[:, pl.ds(0, _TBLK)]
    hi = a_ref[:, pl.ds(_TBLK, _TBLK)]
    o_ref[...] = jnp.concatenate([lo.T, hi.T], axis=1)


@functools.partial(
    pl.kernel,
    mesh=_mesh,
    out_type=jax.ShapeDtypeStruct((_B * 16,), jnp.float32),
    scratch_types=[
        pltpu.VMEM((_NCHUNK, _CHUNK), jnp.int32),
        pltpu.VMEM((_NCHUNK, _CHUNK), jnp.int32),
        pltpu.VMEM((_NBUF, _CHUNK, 2 * _D), jnp.float32),
        pltpu.VMEM((_BPW * 16,), jnp.float32),
        pltpu.SemaphoreType.DMA,
    ],
)
def _sc_pair_lorentz(idx_hbm, w128_hbm, out_hbm, idx_v, row_v, buf_v, t_v,
                     sem):
    wid = lax.axis_index("s") * _NC + lax.axis_index("c")
    pltpu.sync_copy(idx_hbm.at[wid], idx_v)

    for c in range(_NCHUNK):
        for j in range(_CHUNK // 16):
            sl = pl.ds(j * 16, 16)
            row_v[c, sl] = idx_v[c, sl] & jnp.int32(0x7FFFF)

    def fire(c):
        return pltpu.async_copy(
            w128_hbm.at[row_v.at[c]], buf_v.at[c % _NBUF], sem
        )

    copies = {c: fire(c) for c in range(min(2, _NCHUNK))}

    lane = lax.iota(jnp.int32, 16)
    sgn = jnp.where(lane == 0, jnp.float32(1.0), jnp.float32(-1.0))

    for c in range(_NCHUNK):
        if c + 2 < _NCHUNK:
            copies[c + 2] = fire(c + 2)
        copies[c].wait()
        bu = buf_v.at[c % _NBUF]

        def grp(g, carry, c=c, bu=bu):
            ia = idx_v[c, pl.ds(g * 32, 16)]
            ib = idx_v[c, pl.ds(g * 32 + 16, 16)]
            for kk in range(16):
                k = g * 16 + kk
                src = ia if kk < 8 else ib
                iu = src[(2 * kk) % 16]
                iv = src[(2 * kk + 1) % 16]
                offu = ((iu >> 19) & 1) * _D
                offv = ((iv >> 19) & 1) * _D
                t = (
                    bu[2 * k, pl.ds(offu, 16)]
                    * bu[2 * k + 1, pl.ds(offv, 16)]
                    * sgn
                )
                for q in range(1, 4):
                    t = t - (
                        bu[2 * k, pl.ds(offu + q * 16, 16)]
                        * bu[2 * k + 1, pl.ds(offv + q * 16, 16)]
                    )
                t_v[pl.ds(c * 1024 + g * 256 + kk * 16, 16)] = t
            return carry

        lax.fori_loop(0, 4, grp, 0)

    pltpu.sync_copy(t_v, out_hbm.at[pl.ds(wid * _BPW * 16, _BPW * 16)])


def _fold_acosh_body(x_ref, s_ref, o_ref):
    x = x_ref[...]
    col = jax.lax.broadcasted_iota(jnp.int32, (128, 8), 0)
    grp = jax.lax.broadcasted_iota(jnp.int32, (128, 8), 1)
    m = jnp.where(col // 16 == grp, jnp.float32(1.0), jnp.float32(0.0))
    arg = jax.lax.dot_general(
        x, m, (((1,), (0,)), ((), ())),
        precision=jax.lax.Precision.HIGHEST,
        preferred_element_type=jnp.float32,
    )
    arg = jnp.maximum(arg, jnp.float32(1.0 + 1e-9))
    scale = jnp.exp(s_ref[0])
    o_ref[...] = jnp.log(arg + jnp.sqrt((arg - 1.0) * (arg + 1.0))) * scale


@jax.jit
def kernel(idx, w, scale_log):
    e = idx.astype(jnp.int32)
    q = e >> 11
    packed = ((q >> 1) * _TBLK + (e & 2047)) | ((q & 1) << 19)
    idx32 = packed.reshape(_NW, _NCHUNK, _CHUNK)

    wt = w.T
    w128 = pl.pallas_call(
        _transpose_body,
        grid=(_TGRID,),
        in_specs=[pl.BlockSpec((_D, 2 * _TBLK), lambda i: (0, i))],
        out_specs=pl.BlockSpec((_TBLK, 2 * _D), lambda i: (i, 0)),
        out_shape=jax.ShapeDtypeStruct((_HROWS, 2 * _D), jnp.float32),
    )(wt)

    partials = _sc_pair_lorentz(idx32, w128)
    out = pl.pallas_call(
        _fold_acosh_body,
        out_shape=jax.ShapeDtypeStruct((_B // 8, 8), jnp.float32),
        in_specs=[
            pl.BlockSpec(memory_space=pltpu.VMEM),
            pl.BlockSpec(memory_space=pltpu.SMEM),
        ],
        out_specs=pl.BlockSpec(memory_space=pltpu.VMEM),
    )(partials.reshape(_B // 8, 128), scale_log)
    return out.reshape(_B)


# speedup vs baseline: 2.5118x; 1.4655x over previous
import functools

import jax
import jax.numpy as jnp
from jax import lax
from jax.experimental import pallas as pl
from jax.experimental.pallas import tpu as pltpu
from jax.experimental.pallas import tpu_sc as plsc

_D = 64
_B = 16384
_NC = 2
_NS = 16
_NW = _NC * _NS
_BPW = _B // _NW
_RPW = 2 * _BPW
_CHUNK = 128
_NCHUNK = _RPW // _CHUNK
_NBUF = 3

_mesh = plsc.VectorSubcoreMesh(core_axis_name="c", subcore_axis_name="s")


@functools.partial(
    pl.kernel,
    mesh=_mesh,
    out_type=jax.ShapeDtypeStruct((_B * 16,), jnp.float32),
    scratch_types=[
        pltpu.VMEM((_NCHUNK, _CHUNK), jnp.int32),
        pltpu.VMEM((_NBUF, _CHUNK, 1, _D), jnp.float32),
        pltpu.VMEM((_BPW * 16,), jnp.float32),
        pltpu.SemaphoreType.DMA,
    ],
)
def _sc_pair_lorentz(idx_hbm, w3_hbm, out_hbm, idx_v, buf_v, t_v, sem):
    wid = lax.axis_index("s") * _NC + lax.axis_index("c")
    pltpu.sync_copy(idx_hbm.at[wid], idx_v)

    def fire(c):
        return pltpu.async_copy(
            w3_hbm.at[idx_v.at[c]], buf_v.at[c % _NBUF], sem
        )

    copies = {c: fire(c) for c in range(min(2, _NCHUNK))}

    lane = lax.iota(jnp.int32, 16)
    sgn = jnp.where(lane == 0, jnp.float32(1.0), jnp.float32(-1.0))

    for c in range(_NCHUNK):
        if c + 2 < _NCHUNK:
            copies[c + 2] = fire(c + 2)
        copies[c].wait()
        bu = buf_v.at[c % _NBUF]

        def grp(g, carry, c=c, bu=bu):
            for kk in range(16):
                k = g * 16 + kk
                t = (
                    bu[2 * k, 0, pl.ds(0, 16)]
                    * bu[2 * k + 1, 0, pl.ds(0, 16)]
                    * sgn
                )
                for q in range(1, 4):
                    t = t - (
                        bu[2 * k, 0, pl.ds(q * 16, 16)]
                        * bu[2 * k + 1, 0, pl.ds(q * 16, 16)]
                    )
                t_v[pl.ds(c * 1024 + g * 256 + kk * 16, 16)] = t
            return carry

        lax.fori_loop(0, 4, grp, 0)

    pltpu.sync_copy(t_v, out_hbm.at[pl.ds(wid * _BPW * 16, _BPW * 16)])


def _fold_acosh_body(x_ref, s_ref, o_ref):
    x = x_ref[...]
    col = jax.lax.broadcasted_iota(jnp.int32, (128, 8), 0)
    grp = jax.lax.broadcasted_iota(jnp.int32, (128, 8), 1)
    m = jnp.where(col // 16 == grp, jnp.float32(1.0), jnp.float32(0.0))
    arg = jax.lax.dot_general(
        x, m, (((1,), (0,)), ((), ())),
        precision=jax.lax.Precision.HIGHEST,
        preferred_element_type=jnp.float32,
    )
    arg = jnp.maximum(arg, jnp.float32(1.0 + 1e-9))
    scale = jnp.exp(s_ref[0])
    o_ref[...] = jnp.log(arg + jnp.sqrt((arg - 1.0) * (arg + 1.0))) * scale


@jax.jit
def kernel(idx, w, scale_log):
    idx32 = idx.astype(jnp.int32).reshape(_NW, _NCHUNK, _CHUNK)
    w3 = w[:, None, :]
    partials = _sc_pair_lorentz(idx32, w3)
    out = pl.pallas_call(
        _fold_acosh_body,
        out_shape=jax.ShapeDtypeStruct((_B // 8, 8), jnp.float32),
        in_specs=[
            pl.BlockSpec(memory_space=pltpu.VMEM),
            pl.BlockSpec(memory_space=pltpu.SMEM),
        ],
        out_specs=pl.BlockSpec(memory_space=pltpu.VMEM),
    )(partials.reshape(_B // 8, 128), scale_log)
    return out.reshape(_B)
